# baseline (speedup 1.0000x reference)
import functools

import jax
import jax.numpy as jnp
import numpy as np
from jax.experimental import pallas as pl
from jax.experimental.pallas import tpu as pltpu

B = 2
N = 8192
DIM = 256
HIDDEN = 128
NGROUPS = 8
K = 16
RATE = 4
M = N // RATE


def _fps_body(x_ref, idx_ref):
    lane = jax.lax.broadcasted_iota(jnp.int32, (1, N), 1)
    x0 = x_ref[0]
    x1 = x_ref[1]
    idx_ref[0, 0, 0] = 0
    idx_ref[1, 0, 0] = 0

    def body(i, carry):
        d0s, d1s, l0, l1 = carry
        oh0 = jnp.where(lane == l0, 1.0, 0.0)
        oh1 = jnp.where(lane == l1, 1.0, 0.0)
        dn = (((1,), (1,)), ((), ()))
        p0 = jax.lax.dot_general(x0, oh0, dn,
                                 preferred_element_type=jnp.float32)
        p1 = jax.lax.dot_general(x1, oh1, dn,
                                 preferred_element_type=jnp.float32)
        d0s = jnp.minimum(d0s, jnp.sum((x0 - p0) ** 2, axis=0, keepdims=True))
        d1s = jnp.minimum(d1s, jnp.sum((x1 - p1) ** 2, axis=0, keepdims=True))
        m0 = jnp.max(d0s)
        m1 = jnp.max(d1s)
        n0 = jnp.min(jnp.where(d0s == m0, lane, N)).astype(jnp.int32)
        n1 = jnp.min(jnp.where(d1s == m1, lane, N)).astype(jnp.int32)
        idx_ref[0, 0, i] = n0
        idx_ref[1, 0, i] = n1
        return d0s, d1s, n0, n1

    dists0 = jnp.full((1, N), jnp.inf, jnp.float32)
    jax.lax.fori_loop(1, M, body,
                      (dists0, dists0, jnp.int32(0), jnp.int32(0)))


def _fps_pallas(xyzs):
    fps_idx = pl.pallas_call(
        _fps_body,
        grid=(1,),
        in_specs=[pl.BlockSpec((B, 3, N), lambda i: (0, 0, 0))],
        out_specs=pl.BlockSpec((B, 1, M), lambda i: (0, 0, 0),
                               memory_space=pltpu.SMEM),
        out_shape=jax.ShapeDtypeStruct((B, 1, M), jnp.int32),
    )(xyzs)
    fps_idx = fps_idx[:, 0, :]
    sampled_xyzs = _index_points(xyzs, fps_idx)
    return fps_idx, sampled_xyzs


TM = 256
NC = 512
NCHUNKS = N // NC


def _knn_body(xt_ref, s_ref, idx_ref, nnv_ref, nni_ref, d_ref):
    t = pl.program_id(1)
    qq = jax.lax.broadcasted_iota(jnp.int32, (1, TM), 1) + t * TM

    for ci in range(NCHUNKS):
        d = jnp.zeros((NC, TM), jnp.float32)
        for c in range(3):
            xc = xt_ref[0, pl.ds(ci * NC, NC), pl.ds(c, 1)]
            sc = s_ref[0, pl.ds(c, 1), :]
            d = d + (xc - sc) ** 2
        d_ref[pl.ds(ci * NC, NC), :] = d
        rmin = jnp.min(d, axis=1, keepdims=True)
        rarg = jnp.min(jnp.where(d == rmin, qq, M),
                       axis=1, keepdims=True).astype(jnp.int32)

        @pl.when(t == 0)
        def _():
            nnv_ref[0, pl.ds(ci * NC, NC), :] = rmin
            nni_ref[0, pl.ds(ci * NC, NC), :] = rarg

        @pl.when(t != 0)
        def _():
            cur_v = nnv_ref[0, pl.ds(ci * NC, NC), :]
            cur_i = nni_ref[0, pl.ds(ci * NC, NC), :]
            upd = rmin < cur_v
            nnv_ref[0, pl.ds(ci * NC, NC), :] = jnp.where(upd, rmin, cur_v)
            nni_ref[0, pl.ds(ci * NC, NC), :] = jnp.where(upd, rarg, cur_i)

    last_v = jnp.full((1, TM), -jnp.inf, jnp.float32)
    last_i = jnp.full((1, TM), -1, jnp.int32)
    for k in range(K):
        best_v = jnp.full((1, TM), jnp.inf, jnp.float32)
        best_i = jnp.full((1, TM), N, jnp.int32)
        for ci in range(NCHUNKS):
            v = d_ref[pl.ds(ci * NC, NC), :]
            jj = (jax.lax.broadcasted_iota(jnp.int32, (NC, TM), 0)
                  + ci * NC)
            excl = (v < last_v) | ((v == last_v) & (jj <= last_i))
            vv = jnp.where(excl, jnp.inf, v)
            cm = jnp.min(vv, axis=0, keepdims=True)
            ca = jnp.min(jnp.where(vv == cm, jj, N),
                         axis=0, keepdims=True).astype(jnp.int32)
            upd = cm < best_v
            best_i = jnp.where(upd, ca, best_i)
            best_v = jnp.where(upd, cm, best_v)
        idx_ref[0, pl.ds(k, 1), :] = best_i
        last_v, last_i = best_v, best_i


def _knn_pallas(xyzs_t, sampled):
    knn_km, _nnv, nni = pl.pallas_call(
        _knn_body,
        grid=(B, M // TM),
        in_specs=[
            pl.BlockSpec((1, N, 3), lambda b, t: (b, 0, 0)),
            pl.BlockSpec((1, 3, TM), lambda b, t: (b, 0, t)),
        ],
        out_specs=[
            pl.BlockSpec((1, K, TM), lambda b, t: (b, 0, t)),
            pl.BlockSpec((1, N, 1), lambda b, t: (b, 0, 0)),
            pl.BlockSpec((1, N, 1), lambda b, t: (b, 0, 0)),
        ],
        out_shape=[
            jax.ShapeDtypeStruct((B, K, M), jnp.int32),
            jax.ShapeDtypeStruct((B, N, 1), jnp.float32),
            jax.ShapeDtypeStruct((B, N, 1), jnp.int32),
        ],
        scratch_shapes=[pltpu.VMEM((N, TM), jnp.float32)],
    )(xyzs_t, sampled)
    knn_idx = jnp.transpose(knn_km, (0, 2, 1))
    nn_idx = nni[:, :, 0]
    return knn_idx, nn_idx


from jax import lax as _lax
from jax.experimental.pallas import tpu_sc as plsc

MK_ = M * K
BMK = B * MK_
BN = N * B
GC = 128


CW = 384


def _sc_gather(comb, f2t, kidx, qidx):
    info = plsc.get_sparse_core_info()
    nw = info.num_cores * info.num_subcores
    rows_pw = BMK // nw
    nch = rows_pw // GC
    q_pw = (B * M) // nw
    mesh = plsc.VectorSubcoreMesh(core_axis_name="c", subcore_axis_name="s")

    @functools.partial(
        pl.kernel, mesh=mesh,
        out_type=[
            jax.ShapeDtypeStruct((BMK, CW), jnp.float32),
            jax.ShapeDtypeStruct((B * M, DIM), jnp.float32),
        ],
        scratch_types=[
            pltpu.VMEM((GC,), jnp.int32),
            pltpu.VMEM((GC, CW), jnp.float32),
            pltpu.VMEM((q_pw,), jnp.int32),
            pltpu.VMEM((q_pw, DIM), jnp.float32),
            pltpu.SemaphoreType.DMA,
        ],
    )
    def gather_k(comb_h, f2t_h, kidx_h, qidx_h, combg_h, qf_h,
                 idx_v, cbuf, qidx_v, qbuf, sem):
        wid = _lax.axis_index("s") * info.num_cores + _lax.axis_index("c")
        base0 = wid * rows_pw
        for ci in range(nch):
            base = base0 + ci * GC
            pltpu.sync_copy(kidx_h.at[pl.ds(base, GC)], idx_v)
            pltpu.async_copy(comb_h.at[idx_v], cbuf, sem).wait()
            pltpu.sync_copy(cbuf, combg_h.at[pl.ds(base, GC)])
        qbase = wid * q_pw
        pltpu.sync_copy(qidx_h.at[pl.ds(qbase, q_pw)], qidx_v)
        pltpu.async_copy(f2t_h.at[qidx_v], qbuf, sem).wait()
        pltpu.sync_copy(qbuf, qf_h.at[pl.ds(qbase, q_pw)])

    return gather_k(comb, f2t, kidx, qidx)


PT = 512
QT = 256
QTK = QT * K
MK = M * K
FMAX = float(np.finfo(np.float32).max)
RSQRTH = 1.0 / np.sqrt(HIDDEN)


def _proj_body(xt_ref, prew_ref, preb_ref, wk_ref, wkb_ref, wv_ref, wvb_ref,
               f2_ref, key_ref, val_ref):
    x = xt_ref[0]
    f2 = jnp.dot(x, prew_ref[...],
                 preferred_element_type=jnp.float32) + preb_ref[...]
    f2_ref[0] = f2
    key_ref[0] = jnp.dot(f2, wk_ref[...],
                         preferred_element_type=jnp.float32) + wkb_ref[...]
    val_ref[0] = jnp.dot(f2, wv_ref[...],
                         preferred_element_type=jnp.float32) + wvb_ref[...]


def _proj_pallas(feats_t, p):
    w2 = lambda name: p[name].T
    b2 = lambda name: p[name].reshape(1, -1)
    wspec = lambda a: pl.BlockSpec(a.shape, lambda b, t: (0,) * a.ndim)
    args = [feats_t, w2('pre_w'), b2('pre_b'), w2('wk_w'), b2('wk_b'),
            w2('wv_w'), b2('wv_b')]
    return pl.pallas_call(
        _proj_body,
        grid=(B, N // PT),
        in_specs=[pl.BlockSpec((1, PT, DIM), lambda b, t: (b, t, 0))]
        + [wspec(a) for a in args[1:]],
        out_specs=[
            pl.BlockSpec((1, PT, DIM), lambda b, t: (b, t, 0)),
            pl.BlockSpec((1, PT, HIDDEN), lambda b, t: (b, t, 0)),
            pl.BlockSpec((1, PT, HIDDEN), lambda b, t: (b, t, 0)),
        ],
        out_shape=[
            jax.ShapeDtypeStruct((B, N, DIM), jnp.float32),
            jax.ShapeDtypeStruct((B, N, HIDDEN), jnp.float32),
            jax.ShapeDtypeStruct((B, N, HIDDEN), jnp.float32),
        ],
    )(*args)


def _rep_k(x):
    return jnp.reshape(
        jnp.broadcast_to(x[:, None, :], (QT, K, x.shape[1])),
        (QTK, x.shape[1]))


def _geom(kx, sx):
    sxr = _rep_k(sx)
    dg = sxr - kx
    offset = kx - sxr
    dist = jnp.sqrt(jnp.sum(offset ** 2, axis=1, keepdims=True) + 1e-12)
    direction = offset / jnp.maximum(dist, 1e-12)
    local = jnp.concatenate([direction, dist], axis=1)
    return dg, local


def _acc_stats(ref, x, first):
    s = jnp.concatenate([jnp.sum(x, axis=0, keepdims=True),
                         jnp.sum(x * x, axis=0, keepdims=True)], axis=0)

    @pl.when(first)
    def _():
        ref[0] = s

    @pl.when(jnp.logical_not(first))
    def _():
        ref[0] += s


def _stats1_body(kx_ref, sx_ref, dw1_ref, db1_ref, pw1_ref, pb1_ref,
                 stpe_ref, ste_ref):
    first = pl.program_id(1) == 0
    dg, local = _geom(kx_ref[0][:, :3], sx_ref[0])
    pe1 = jnp.dot(dg, dw1_ref[...],
                  preferred_element_type=jnp.float32) + db1_ref[...]
    _acc_stats(stpe_ref, pe1, first)
    e1 = jnp.dot(local, pw1_ref[...],
                 preferred_element_type=jnp.float32) + pb1_ref[...]
    _acc_stats(ste_ref, e1, first)


def _stats1_pallas(kx, sxt, p):
    args = [kx, sxt, p['delta_w1'].T, p['delta_b1'].reshape(1, -1),
            p['pe_w1'].T, p['pe_b1'].reshape(1, -1)]
    wspec = lambda a: pl.BlockSpec(a.shape, lambda b, t: (0,) * a.ndim)
    return pl.pallas_call(
        _stats1_body,
        grid=(B, M // QT),
        in_specs=[
            pl.BlockSpec((1, QTK, 16), lambda b, t: (b, t, 0)),
            pl.BlockSpec((1, QT, 3), lambda b, t: (b, t, 0)),
        ] + [wspec(a) for a in args[2:]],
        out_specs=[pl.BlockSpec((1, 2, HIDDEN), lambda b, t: (b, 0, 0))] * 2,
        out_shape=[jax.ShapeDtypeStruct((B, 2, HIDDEN), jnp.float32)] * 2,
    )(*args)


def _stats2_body(kx_ref, sx_ref, qf_ref, keyg_ref,
                 dw1_ref, db1_ref, dsc_ref, dsh_ref, dw2_ref, db2_ref,
                 pw1_ref, pb1_ref, psc_ref, psh_ref, pw2_ref, pb2_ref,
                 wq_ref, wqb_ref, gw1_ref, gb1_ref, aw1_ref, ab1_ref,
                 sta_ref, stpa_ref, pos_ref, e2_ref, a1_ref, pa1_ref):
    first = pl.program_id(1) == 0
    dg, local = _geom(kx_ref[0][:, :3], sx_ref[0])
    pe1 = jnp.dot(dg, dw1_ref[...],
                  preferred_element_type=jnp.float32) + db1_ref[...]
    pe1 = jax.nn.relu(pe1 * dsc_ref[0] + dsh_ref[0])
    pos = jnp.dot(pe1, dw2_ref[...],
                  preferred_element_type=jnp.float32) + db2_ref[...]
    pos_ref[0] = pos
    e1 = jnp.dot(local, pw1_ref[...],
                 preferred_element_type=jnp.float32) + pb1_ref[...]
    e1 = jax.nn.relu(e1 * psc_ref[0] + psh_ref[0])
    e2 = jnp.dot(e1, pw2_ref[...],
                 preferred_element_type=jnp.float32) + pb2_ref[...]
    e2_ref[0] = e2
    query = jnp.dot(qf_ref[0], wq_ref[...],
                    preferred_element_type=jnp.float32) + wqb_ref[...]
    a0 = _rep_k(query) - keyg_ref[0] + pos
    a1 = jnp.dot(a0, gw1_ref[...],
                 preferred_element_type=jnp.float32) + gb1_ref[...]
    a1_ref[0] = a1
    _acc_stats(sta_ref, a1, first)
    pa1 = jnp.dot(e2, aw1_ref[...],
                  preferred_element_type=jnp.float32) + ab1_ref[...]
    pa1_ref[0] = pa1
    _acc_stats(stpa_ref, pa1, first)


def _stats2_pallas(kx, sxt, qf, keyg, dcoef, pcoef, p):
    args = [kx, sxt, qf, keyg,
            p['delta_w1'].T, p['delta_b1'].reshape(1, -1),
            dcoef[0], dcoef[1], p['delta_w2'].T, p['delta_b2'].reshape(1, -1),
            p['pe_w1'].T, p['pe_b1'].reshape(1, -1),
            pcoef[0], pcoef[1], p['pe_w2'].T, p['pe_b2'].reshape(1, -1),
            p['wq_w'].T, p['wq_b'].reshape(1, -1),
            p['gamma_w1'].T, p['gamma_b1'].reshape(1, -1),
            p['pa_w1'].T, p['pa_b1'].reshape(1, -1)]

    def spec(a):
        if a.ndim == 3:
            return pl.BlockSpec((1,) + a.shape[1:], lambda b, t: (b, 0, 0))
        return pl.BlockSpec(a.shape, lambda b, t: (0,) * a.ndim)

    return pl.pallas_call(
        _stats2_body,
        grid=(B, M // QT),
        in_specs=[
            pl.BlockSpec((1, QTK, 16), lambda b, t: (b, t, 0)),
            pl.BlockSpec((1, QT, 3), lambda b, t: (b, t, 0)),
            pl.BlockSpec((1, QT, DIM), lambda b, t: (b, t, 0)),
            pl.BlockSpec((1, QTK, HIDDEN), lambda b, t: (b, t, 0)),
        ] + [spec(a) for a in args[4:]],
        out_specs=[pl.BlockSpec((1, 2, HIDDEN), lambda b, t: (b, 0, 0))] * 2
        + [
            pl.BlockSpec((1, QTK, HIDDEN), lambda b, t: (b, t, 0)),
            pl.BlockSpec((1, QTK, DIM), lambda b, t: (b, t, 0)),
            pl.BlockSpec((1, QTK, HIDDEN), lambda b, t: (b, t, 0)),
            pl.BlockSpec((1, QTK, HIDDEN), lambda b, t: (b, t, 0)),
        ],
        out_shape=[jax.ShapeDtypeStruct((B, 2, HIDDEN), jnp.float32)] * 2
        + [
            jax.ShapeDtypeStruct((B, MK, HIDDEN), jnp.float32),
            jax.ShapeDtypeStruct((B, MK, DIM), jnp.float32),
            jax.ShapeDtypeStruct((B, MK, HIDDEN), jnp.float32),
            jax.ShapeDtypeStruct((B, MK, HIDDEN), jnp.float32),
        ],
    )(*args)


def _softmax_k(a):
    a3 = a.reshape(QT, K, a.shape[1])
    amax = jnp.max(a3, axis=1, keepdims=True)
    ae = jnp.exp(a3 - amax)
    return ae / jnp.sum(ae, axis=1, keepdims=True)


def _final_body(a1_ref, pa1_ref, pos_ref, e2_ref, valg_ref, qf_ref,
                msk_ref, embd_ref,
                gsc_ref, gsh_ref, gw2_ref, gb2_ref, postw_ref, postb_ref,
                asc_ref, ash_ref, aw2_ref, ab2_ref, emb_ref):
    msk = msk_ref[0] > 0
    pos = pos_ref[0]
    a = jax.nn.relu(a1_ref[0] * gsc_ref[0] + gsh_ref[0])
    a = jnp.dot(a, gw2_ref[...],
                preferred_element_type=jnp.float32) + gb2_ref[...]
    a = a * RSQRTH
    a = jnp.where(msk, a, -FMAX)
    aw = _softmax_k(a)
    vpe = (valg_ref[0] + pos).reshape(QT, K, HIDDEN)
    res = jnp.sum(aw * vpe, axis=1)
    emb_f = jnp.dot(res, postw_ref[...],
                    preferred_element_type=jnp.float32) + postb_ref[...]
    emb_f = emb_f + qf_ref[0]
    pa = jax.nn.relu(pa1_ref[0] * asc_ref[0] + ash_ref[0])
    pa2 = jnp.dot(pa, aw2_ref[...],
                  preferred_element_type=jnp.float32) + ab2_ref[...]
    pa2 = jnp.where(msk, pa2, -FMAX)
    pw = _softmax_k(pa2)
    e3 = e2_ref[0].reshape(QT, K, DIM)
    emb_p = jnp.sum(e3 * pw, axis=1)
    emb_ref[0] = emb_f + emb_p + embd_ref[0]


def _final_pallas(a1, pa1, pos, e2, valg, qf, maskrow, embd, gcoef, acoef, p):
    args = [a1, pa1, pos, e2, valg, qf, maskrow, embd,
            gcoef[0], gcoef[1], p['gamma_w2'].T, p['gamma_b2'].reshape(1, -1),
            p['post_w'].T, p['post_b'].reshape(1, -1),
            acoef[0], acoef[1], p['pa_w2'].T, p['pa_b2'].reshape(1, -1)]

    def spec(a):
        if a.ndim == 3:
            return pl.BlockSpec((1,) + a.shape[1:], lambda b, t: (b, 0, 0))
        return pl.BlockSpec(a.shape, lambda b, t: (0,) * a.ndim)

    return pl.pallas_call(
        _final_body,
        grid=(B, M // QT),
        in_specs=[
            pl.BlockSpec((1, QTK, HIDDEN), lambda b, t: (b, t, 0)),
            pl.BlockSpec((1, QTK, HIDDEN), lambda b, t: (b, t, 0)),
            pl.BlockSpec((1, QTK, HIDDEN), lambda b, t: (b, t, 0)),
            pl.BlockSpec((1, QTK, DIM), lambda b, t: (b, t, 0)),
            pl.BlockSpec((1, QTK, HIDDEN), lambda b, t: (b, t, 0)),
            pl.BlockSpec((1, QT, DIM), lambda b, t: (b, t, 0)),
            pl.BlockSpec((1, QTK, 1), lambda b, t: (b, t, 0)),
            pl.BlockSpec((1, QT, DIM), lambda b, t: (b, t, 0)),
        ] + [spec(a) for a in args[8:]],
        out_specs=pl.BlockSpec((1, QT, DIM), lambda b, t: (b, t, 0)),
        out_shape=jax.ShapeDtypeStruct((B, M, DIM), jnp.float32),
    )(*args)


def _den_body(dn_ref, w1_ref, b1_ref, g_ref, gb_ref, w2_ref, b2_ref, out_ref):
    e = jnp.dot(dn_ref[0], w1_ref[...],
                preferred_element_type=jnp.float32) + b1_ref[...]
    lane = jax.lax.broadcasted_iota(jnp.int32, (HIDDEN, NGROUPS), 0)
    grp = jax.lax.broadcasted_iota(jnp.int32, (HIDDEN, NGROUPS), 1)
    G = (lane // (HIDDEN // NGROUPS) == grp).astype(jnp.float32)
    s1 = jnp.sum(e, axis=0, keepdims=True)
    s2 = jnp.sum(e * e, axis=0, keepdims=True)
    cnt = float(M * (HIDDEN // NGROUPS))
    gm = jnp.dot(s1, G, preferred_element_type=jnp.float32) / cnt
    gv = jnp.dot(s2, G, preferred_element_type=jnp.float32) / cnt - gm * gm
    meanl = jax.lax.dot_general(gm, G, (((1,), (1,)), ((), ())),
                                preferred_element_type=jnp.float32)
    rstdl = jax.lax.dot_general(jax.lax.rsqrt(gv + 1e-5), G,
                                (((1,), (1,)), ((), ())),
                                preferred_element_type=jnp.float32)
    xn = (e - meanl) * rstdl * g_ref[...] + gb_ref[...]
    xn = jax.nn.relu(xn)
    out_ref[0] = jnp.dot(xn, w2_ref[...],
                         preferred_element_type=jnp.float32) + b2_ref[...]


def _den_pallas(dn_rows, p):
    args = [dn_rows, p['de_w1'].T, p['de_b1'].reshape(1, -1),
            p['de_g'].reshape(1, -1), p['de_gb'].reshape(1, -1),
            p['de_w2'].T, p['de_b2'].reshape(1, -1)]
    wspec = lambda a: pl.BlockSpec(a.shape, lambda b: (0,) * a.ndim)
    return pl.pallas_call(
        _den_body,
        grid=(B,),
        in_specs=[pl.BlockSpec((1, M, 1), lambda b: (b, 0, 0))]
        + [wspec(a) for a in args[1:]],
        out_specs=pl.BlockSpec((1, M, DIM), lambda b: (b, 0, 0)),
        out_shape=jax.ShapeDtypeStruct((B, M, DIM), jnp.float32),
    )(*args)


def _gn_coeffs(st, gamma, beta, cnt):
    cpg = st.shape[-1] // NGROUPS
    s1 = st[:, 0, :].reshape(B, NGROUPS, cpg).sum(-1)
    s2 = st[:, 1, :].reshape(B, NGROUPS, cpg).sum(-1)
    mean = s1 / cnt
    var = s2 / cnt - mean * mean
    rstd = jax.lax.rsqrt(var + 1e-5)
    meanl = jnp.repeat(mean, cpg, axis=-1)
    rstdl = jnp.repeat(rstd, cpg, axis=-1)
    scale = rstdl * gamma[None, :]
    shift = beta[None, :] - meanl * scale
    return scale[:, None, :], shift[:, None, :]


def _conv1d(x, w, b):
    return jnp.einsum('oc,bcn->bon', w, x) + b[None, :, None]


def _conv2d(x, w, b):
    return jnp.einsum('oc,bcmk->bomk', w, x) + b[None, :, None, None]


def _group_norm(x, g, gamma, beta, eps=1e-5):
    b, c = x.shape[0], x.shape[1]
    rest = x.shape[2:]
    xg = x.reshape((b, g, c // g) + rest)
    axes = tuple(range(2, xg.ndim))
    mean = xg.mean(axis=axes, keepdims=True)
    var = xg.var(axis=axes, keepdims=True)
    xg = (xg - mean) / jnp.sqrt(var + eps)
    x = xg.reshape((b, c) + rest)
    shp = (1, c) + (1,) * len(rest)
    return x * gamma.reshape(shp) + beta.reshape(shp)


def _index_points(points, idx):
    return jax.vmap(lambda p, i: p[:, i])(points, idx)


def _knn(k, ref_tr, query_tr):
    d = jnp.sum((query_tr[:, :, None, :] - ref_tr[:, None, :, :]) ** 2, axis=-1)
    _, idx = jax.lax.top_k(-d, k)
    return idx


def _nearby(a_xyzs, b_xyzs, k):
    m = a_xyzs.shape[2]
    a_tr = jnp.transpose(a_xyzs, (0, 2, 1))
    b_tr = jnp.transpose(b_xyzs, (0, 2, 1))
    nn_idx = _knn(1, a_tr, b_tr)
    knn_idx = _knn(k, b_tr, a_tr)
    nn_idx_tr = jnp.transpose(nn_idx, (0, 2, 1))
    expect = jnp.arange(m)[None, :, None]
    actual = _index_points(nn_idx_tr, knn_idx)[:, 0]
    mask = expect == actual
    return mask, knn_idx


def _ptl(q_xyzs, k_xyzs, q_feats, k_feats, v_feats, knn_idx, mask, p):
    knn_xyzs = _index_points(k_xyzs, knn_idx)
    identity = q_feats
    query = _conv1d(q_feats, p['wq_w'], p['wq_b'])
    key = _index_points(_conv1d(k_feats, p['wk_w'], p['wk_b']), knn_idx)
    value = _index_points(_conv1d(v_feats, p['wv_w'], p['wv_b']), knn_idx)
    d = q_xyzs[..., None] - knn_xyzs
    pe = _conv2d(d, p['delta_w1'], p['delta_b1'])
    pe = jax.nn.relu(_group_norm(pe, NGROUPS, p['delta_g'], p['delta_gb']))
    pos_enc = _conv2d(pe, p['delta_w2'], p['delta_b2'])
    a = query[..., None] - key + pos_enc
    a = _conv2d(a, p['gamma_w1'], p['gamma_b1'])
    a = jax.nn.relu(_group_norm(a, NGROUPS, p['gamma_g'], p['gamma_gb']))
    a = _conv2d(a, p['gamma_w2'], p['gamma_b2'])
    a = a / np.sqrt(HIDDEN)
    a = jnp.where(mask[:, None], a, -jnp.finfo(a.dtype).max)
    a = jax.nn.softmax(a, axis=-1)
    res = jnp.einsum('bcmk,bcmk->bcm', a, value + pos_enc)
    return _conv1d(res, p['post_w'], p['post_b']) + identity


def _pos_emb(q_xyzs, k_xyzs, knn_idx, mask, p):
    knn_xyzs = _index_points(k_xyzs, knn_idx)
    offset = knn_xyzs - q_xyzs[..., None]
    dist = jnp.sqrt(jnp.sum(offset ** 2, axis=1, keepdims=True) + 1e-12)
    direction = offset / jnp.maximum(dist, 1e-12)
    local = jnp.concatenate([direction, dist], axis=1)
    e = _conv2d(local, p['pe_w1'], p['pe_b1'])
    e = jax.nn.relu(_group_norm(e, NGROUPS, p['pe_g'], p['pe_gb']))
    e = _conv2d(e, p['pe_w2'], p['pe_b2'])
    a = _conv2d(e, p['pa_w1'], p['pa_b1'])
    a = jax.nn.relu(_group_norm(a, NGROUPS, p['pa_g'], p['pa_gb']))
    a = _conv2d(a, p['pa_w2'], p['pa_b2'])
    a = jnp.where(mask[:, None], a, -jnp.finfo(a.dtype).max)
    a = jax.nn.softmax(a, axis=-1)
    return (e * a).sum(-1)


def _den_emb(dn, p):
    e = _conv1d(dn, p['de_w1'], p['de_b1'])
    e = jax.nn.relu(_group_norm(e, NGROUPS, p['de_g'], p['de_gb']))
    return _conv1d(e, p['de_w2'], p['de_b2'])


def kernel(xyzs, feats, params):
    fps_idx, sampled_xyzs = _fps_pallas(xyzs)
    xyzs_t = jnp.transpose(xyzs, (0, 2, 1))
    knn_idx, nn_idx = _knn_pallas(xyzs_t, sampled_xyzs)
    feats_t = jnp.transpose(feats, (0, 2, 1))
    f2t, keyt, valt = _proj_pallas(feats_t, params)

    offs = (jnp.arange(B, dtype=jnp.int32) * N)[:, None]
    kidx = (knn_idx.reshape(B, MK) + offs).reshape(BMK)
    qidx = (fps_idx + offs).reshape(B * M)
    nnf = jax.lax.bitcast_convert_type(nn_idx, jnp.float32)
    comb = jnp.concatenate(
        [keyt, valt, xyzs_t, nnf[:, :, None],
         jnp.zeros((B, N, CW - 2 * HIDDEN - 4), jnp.float32)],
        axis=-1).reshape(BN, CW)
    combg, qf_f = _sc_gather(comb, f2t.reshape(BN, DIM), kidx, qidx)
    keyg = combg[:, :HIDDEN].reshape(B, MK, HIDDEN)
    valg = combg[:, HIDDEN:2 * HIDDEN].reshape(B, MK, HIDDEN)
    kx = combg[:, 2 * HIDDEN:2 * HIDDEN + 16].reshape(B, MK, 16)
    qf = qf_f.reshape(B, M, DIM)
    actual = jax.lax.bitcast_convert_type(
        combg[:, 2 * HIDDEN + 3], jnp.int32).reshape(B, M, K)
    mask = jnp.arange(M)[None, :, None] == actual
    downsample_num = mask.astype(jnp.float32).sum(-1)
    sxt = jnp.transpose(sampled_xyzs, (0, 2, 1))

    cnt = float((HIDDEN // NGROUPS) * MK)
    st_pe, st_e = _stats1_pallas(kx, sxt, params)
    dcoef = _gn_coeffs(st_pe, params['delta_g'], params['delta_gb'], cnt)
    pcoef = _gn_coeffs(st_e, params['pe_g'], params['pe_gb'], cnt)
    st_a, st_pa, pos, e2, a1, pa1 = _stats2_pallas(
        kx, sxt, qf, keyg, dcoef, pcoef, params)
    gcoef = _gn_coeffs(st_a, params['gamma_g'], params['gamma_gb'], cnt)
    acoef = _gn_coeffs(st_pa, params['pa_g'], params['pa_gb'], cnt)

    maskrow = mask.reshape(B, MK, 1).astype(jnp.float32)
    embd = _den_pallas(downsample_num[:, :, None], params)
    emb = _final_pallas(a1, pa1, pos, e2, valg, qf, maskrow, embd,
                        gcoef, acoef, params)
    embedded_points = jnp.transpose(emb, (0, 2, 1))
    return sampled_xyzs, embedded_points, downsample_num


# speedup vs baseline: 1.0583x; 1.0583x over previous
import functools

import jax
import jax.numpy as jnp
import numpy as np
from jax.experimental import pallas as pl
from jax.experimental.pallas import tpu as pltpu

B = 2
N = 8192
DIM = 256
HIDDEN = 128
NGROUPS = 8
K = 16
RATE = 4
M = N // RATE


def _fps_body(x_ref, idx_ref):
    lane = jax.lax.broadcasted_iota(jnp.int32, (1, N), 1)
    x0 = x_ref[0]
    x1 = x_ref[1]
    idx_ref[0, 0, 0] = 0
    idx_ref[1, 0, 0] = 0

    def body(i, carry):
        d0s, d1s, l0, l1 = carry
        oh0 = jnp.where(lane == l0, 1.0, 0.0)
        oh1 = jnp.where(lane == l1, 1.0, 0.0)
        dn = (((1,), (1,)), ((), ()))
        p0 = jax.lax.dot_general(x0, oh0, dn,
                                 preferred_element_type=jnp.float32)
        p1 = jax.lax.dot_general(x1, oh1, dn,
                                 preferred_element_type=jnp.float32)
        d0s = jnp.minimum(d0s, jnp.sum((x0 - p0) ** 2, axis=0, keepdims=True))
        d1s = jnp.minimum(d1s, jnp.sum((x1 - p1) ** 2, axis=0, keepdims=True))
        m0 = jnp.max(d0s)
        m1 = jnp.max(d1s)
        n0 = jnp.min(jnp.where(d0s == m0, lane, N)).astype(jnp.int32)
        n1 = jnp.min(jnp.where(d1s == m1, lane, N)).astype(jnp.int32)
        idx_ref[0, 0, i] = n0
        idx_ref[1, 0, i] = n1
        return d0s, d1s, n0, n1

    dists0 = jnp.full((1, N), jnp.inf, jnp.float32)
    jax.lax.fori_loop(1, M, body,
                      (dists0, dists0, jnp.int32(0), jnp.int32(0)))


def _fps_pallas(xyzs):
    fps_idx = pl.pallas_call(
        _fps_body,
        grid=(1,),
        in_specs=[pl.BlockSpec((B, 3, N), lambda i: (0, 0, 0))],
        out_specs=pl.BlockSpec((B, 1, M), lambda i: (0, 0, 0),
                               memory_space=pltpu.SMEM),
        out_shape=jax.ShapeDtypeStruct((B, 1, M), jnp.int32),
    )(xyzs)
    fps_idx = fps_idx[:, 0, :]
    sampled_xyzs = _index_points(xyzs, fps_idx)
    return fps_idx, sampled_xyzs


TM = 512
NC = 1024
NCHUNKS = N // NC


def _knn_body(xt_ref, s_ref, idx_ref, nnv_ref, nni_ref, d_ref):
    t = pl.program_id(1)
    qq = jax.lax.broadcasted_iota(jnp.int32, (1, TM), 1) + t * TM

    for ci in range(NCHUNKS):
        d = jnp.zeros((NC, TM), jnp.float32)
        for c in range(3):
            xc = xt_ref[0, pl.ds(ci * NC, NC), pl.ds(c, 1)]
            sc = s_ref[0, pl.ds(c, 1), :]
            d = d + (xc - sc) ** 2
        d_ref[pl.ds(ci * NC, NC), :] = d
        rmin = jnp.min(d, axis=1, keepdims=True)
        rarg = jnp.min(jnp.where(d == rmin, qq, M),
                       axis=1, keepdims=True).astype(jnp.int32)

        @pl.when(t == 0)
        def _():
            nnv_ref[0, pl.ds(ci * NC, NC), :] = rmin
            nni_ref[0, pl.ds(ci * NC, NC), :] = rarg

        @pl.when(t != 0)
        def _():
            cur_v = nnv_ref[0, pl.ds(ci * NC, NC), :]
            cur_i = nni_ref[0, pl.ds(ci * NC, NC), :]
            upd = rmin < cur_v
            nnv_ref[0, pl.ds(ci * NC, NC), :] = jnp.where(upd, rmin, cur_v)
            nni_ref[0, pl.ds(ci * NC, NC), :] = jnp.where(upd, rarg, cur_i)

    last_v = jnp.full((1, TM), -jnp.inf, jnp.float32)
    last_i = jnp.full((1, TM), -1, jnp.int32)
    for k in range(K):
        best_v = jnp.full((1, TM), jnp.inf, jnp.float32)
        best_i = jnp.full((1, TM), N, jnp.int32)
        for ci in range(NCHUNKS):
            v = d_ref[pl.ds(ci * NC, NC), :]
            jj = (jax.lax.broadcasted_iota(jnp.int32, (NC, TM), 0)
                  + ci * NC)
            excl = (v < last_v) | ((v == last_v) & (jj <= last_i))
            vv = jnp.where(excl, jnp.inf, v)
            cm = jnp.min(vv, axis=0, keepdims=True)
            ca = jnp.min(jnp.where(vv == cm, jj, N),
                         axis=0, keepdims=True).astype(jnp.int32)
            upd = cm < best_v
            best_i = jnp.where(upd, ca, best_i)
            best_v = jnp.where(upd, cm, best_v)
        idx_ref[0, pl.ds(k, 1), :] = best_i
        last_v, last_i = best_v, best_i


def _knn_pallas(xyzs_t, sampled):
    knn_km, _nnv, nni = pl.pallas_call(
        _knn_body,
        grid=(B, M // TM),
        in_specs=[
            pl.BlockSpec((1, N, 3), lambda b, t: (b, 0, 0)),
            pl.BlockSpec((1, 3, TM), lambda b, t: (b, 0, t)),
        ],
        out_specs=[
            pl.BlockSpec((1, K, TM), lambda b, t: (b, 0, t)),
            pl.BlockSpec((1, N, 1), lambda b, t: (b, 0, 0)),
            pl.BlockSpec((1, N, 1), lambda b, t: (b, 0, 0)),
        ],
        out_shape=[
            jax.ShapeDtypeStruct((B, K, M), jnp.int32),
            jax.ShapeDtypeStruct((B, N, 1), jnp.float32),
            jax.ShapeDtypeStruct((B, N, 1), jnp.int32),
        ],
        scratch_shapes=[pltpu.VMEM((N, TM), jnp.float32)],
    )(xyzs_t, sampled)
    knn_idx = jnp.transpose(knn_km, (0, 2, 1))
    nn_idx = nni[:, :, 0]
    return knn_idx, nn_idx


from jax import lax as _lax
from jax.experimental.pallas import tpu_sc as plsc

MK_ = M * K
BMK = B * MK_
BN = N * B
GC = 128


CW = 384


def _sc_gather(comb, f2t, kidx, qidx):
    info = plsc.get_sparse_core_info()
    nw = info.num_cores * info.num_subcores
    rows_pw = BMK // nw
    nch = rows_pw // GC
    q_pw = (B * M) // nw
    mesh = plsc.VectorSubcoreMesh(core_axis_name="c", subcore_axis_name="s")

    @functools.partial(
        pl.kernel, mesh=mesh,
        out_type=[
            jax.ShapeDtypeStruct((BMK, CW), jnp.float32),
            jax.ShapeDtypeStruct((B * M, DIM), jnp.float32),
        ],
        scratch_types=[
            pltpu.VMEM((GC,), jnp.int32),
            pltpu.VMEM((GC, CW), jnp.float32),
            pltpu.VMEM((q_pw,), jnp.int32),
            pltpu.VMEM((q_pw, DIM), jnp.float32),
            pltpu.SemaphoreType.DMA,
        ],
    )
    def gather_k(comb_h, f2t_h, kidx_h, qidx_h, combg_h, qf_h,
                 idx_v, cbuf, qidx_v, qbuf, sem):
        wid = _lax.axis_index("s") * info.num_cores + _lax.axis_index("c")
        base0 = wid * rows_pw
        for ci in range(nch):
            base = base0 + ci * GC
            pltpu.sync_copy(kidx_h.at[pl.ds(base, GC)], idx_v)
            pltpu.async_copy(comb_h.at[idx_v], cbuf, sem).wait()
            pltpu.sync_copy(cbuf, combg_h.at[pl.ds(base, GC)])
        qbase = wid * q_pw
        pltpu.sync_copy(qidx_h.at[pl.ds(qbase, q_pw)], qidx_v)
        pltpu.async_copy(f2t_h.at[qidx_v], qbuf, sem).wait()
        pltpu.sync_copy(qbuf, qf_h.at[pl.ds(qbase, q_pw)])

    return gather_k(comb, f2t, kidx, qidx)


PT = 512
QT = 256
QTK = QT * K
MK = M * K
FMAX = float(np.finfo(np.float32).max)
RSQRTH = 1.0 / np.sqrt(HIDDEN)


def _proj_body(xt_ref, prew_ref, preb_ref, wk_ref, wkb_ref, wv_ref, wvb_ref,
               f2_ref, key_ref, val_ref):
    x = xt_ref[0]
    f2 = jnp.dot(x, prew_ref[...],
                 preferred_element_type=jnp.float32) + preb_ref[...]
    f2_ref[0] = f2
    key_ref[0] = jnp.dot(f2, wk_ref[...],
                         preferred_element_type=jnp.float32) + wkb_ref[...]
    val_ref[0] = jnp.dot(f2, wv_ref[...],
                         preferred_element_type=jnp.float32) + wvb_ref[...]


def _proj_pallas(feats_t, p):
    w2 = lambda name: p[name].T
    b2 = lambda name: p[name].reshape(1, -1)
    wspec = lambda a: pl.BlockSpec(a.shape, lambda b, t: (0,) * a.ndim)
    args = [feats_t, w2('pre_w'), b2('pre_b'), w2('wk_w'), b2('wk_b'),
            w2('wv_w'), b2('wv_b')]
    return pl.pallas_call(
        _proj_body,
        grid=(B, N // PT),
        in_specs=[pl.BlockSpec((1, PT, DIM), lambda b, t: (b, t, 0))]
        + [wspec(a) for a in args[1:]],
        out_specs=[
            pl.BlockSpec((1, PT, DIM), lambda b, t: (b, t, 0)),
            pl.BlockSpec((1, PT, HIDDEN), lambda b, t: (b, t, 0)),
            pl.BlockSpec((1, PT, HIDDEN), lambda b, t: (b, t, 0)),
        ],
        out_shape=[
            jax.ShapeDtypeStruct((B, N, DIM), jnp.float32),
            jax.ShapeDtypeStruct((B, N, HIDDEN), jnp.float32),
            jax.ShapeDtypeStruct((B, N, HIDDEN), jnp.float32),
        ],
    )(*args)


def _rep_k(x):
    return jnp.reshape(
        jnp.broadcast_to(x[:, None, :], (QT, K, x.shape[1])),
        (QTK, x.shape[1]))


def _geom(kx, sx):
    sxr = _rep_k(sx)
    dg = sxr - kx
    offset = kx - sxr
    dist = jnp.sqrt(jnp.sum(offset ** 2, axis=1, keepdims=True) + 1e-12)
    direction = offset / jnp.maximum(dist, 1e-12)
    local = jnp.concatenate([direction, dist], axis=1)
    return dg, local


def _acc_stats(ref, x, first):
    s = jnp.concatenate([jnp.sum(x, axis=0, keepdims=True),
                         jnp.sum(x * x, axis=0, keepdims=True)], axis=0)

    @pl.when(first)
    def _():
        ref[0] = s

    @pl.when(jnp.logical_not(first))
    def _():
        ref[0] += s


def _stats1_body(kx_ref, sx_ref, dw1_ref, db1_ref, pw1_ref, pb1_ref,
                 stpe_ref, ste_ref):
    first = pl.program_id(1) == 0
    dg, local = _geom(kx_ref[0][:, :3], sx_ref[0])
    pe1 = jnp.dot(dg, dw1_ref[...],
                  preferred_element_type=jnp.float32) + db1_ref[...]
    _acc_stats(stpe_ref, pe1, first)
    e1 = jnp.dot(local, pw1_ref[...],
                 preferred_element_type=jnp.float32) + pb1_ref[...]
    _acc_stats(ste_ref, e1, first)


def _stats1_pallas(kx, sxt, p):
    args = [kx, sxt, p['delta_w1'].T, p['delta_b1'].reshape(1, -1),
            p['pe_w1'].T, p['pe_b1'].reshape(1, -1)]
    wspec = lambda a: pl.BlockSpec(a.shape, lambda b, t: (0,) * a.ndim)
    return pl.pallas_call(
        _stats1_body,
        grid=(B, M // QT),
        in_specs=[
            pl.BlockSpec((1, QTK, 16), lambda b, t: (b, t, 0)),
            pl.BlockSpec((1, QT, 3), lambda b, t: (b, t, 0)),
        ] + [wspec(a) for a in args[2:]],
        out_specs=[pl.BlockSpec((1, 2, HIDDEN), lambda b, t: (b, 0, 0))] * 2,
        out_shape=[jax.ShapeDtypeStruct((B, 2, HIDDEN), jnp.float32)] * 2,
    )(*args)


def _stats2_body(kx_ref, sx_ref, qf_ref, keyg_ref,
                 dw1_ref, db1_ref, dsc_ref, dsh_ref, dw2_ref, db2_ref,
                 pw1_ref, pb1_ref, psc_ref, psh_ref, pw2_ref, pb2_ref,
                 wq_ref, wqb_ref, gw1_ref, gb1_ref, aw1_ref, ab1_ref,
                 sta_ref, stpa_ref, pos_ref, e2_ref, a1_ref, pa1_ref):
    first = pl.program_id(1) == 0
    dg, local = _geom(kx_ref[0][:, :3], sx_ref[0])
    pe1 = jnp.dot(dg, dw1_ref[...],
                  preferred_element_type=jnp.float32) + db1_ref[...]
    pe1 = jax.nn.relu(pe1 * dsc_ref[0] + dsh_ref[0])
    pos = jnp.dot(pe1, dw2_ref[...],
                  preferred_element_type=jnp.float32) + db2_ref[...]
    pos_ref[0] = pos
    e1 = jnp.dot(local, pw1_ref[...],
                 preferred_element_type=jnp.float32) + pb1_ref[...]
    e1 = jax.nn.relu(e1 * psc_ref[0] + psh_ref[0])
    e2 = jnp.dot(e1, pw2_ref[...],
                 preferred_element_type=jnp.float32) + pb2_ref[...]
    e2_ref[0] = e2
    query = jnp.dot(qf_ref[0], wq_ref[...],
                    preferred_element_type=jnp.float32) + wqb_ref[...]
    a0 = _rep_k(query) - keyg_ref[0] + pos
    a1 = jnp.dot(a0, gw1_ref[...],
                 preferred_element_type=jnp.float32) + gb1_ref[...]
    a1_ref[0] = a1
    _acc_stats(sta_ref, a1, first)
    pa1 = jnp.dot(e2, aw1_ref[...],
                  preferred_element_type=jnp.float32) + ab1_ref[...]
    pa1_ref[0] = pa1
    _acc_stats(stpa_ref, pa1, first)


def _stats2_pallas(kx, sxt, qf, keyg, dcoef, pcoef, p):
    args = [kx, sxt, qf, keyg,
            p['delta_w1'].T, p['delta_b1'].reshape(1, -1),
            dcoef[0], dcoef[1], p['delta_w2'].T, p['delta_b2'].reshape(1, -1),
            p['pe_w1'].T, p['pe_b1'].reshape(1, -1),
            pcoef[0], pcoef[1], p['pe_w2'].T, p['pe_b2'].reshape(1, -1),
            p['wq_w'].T, p['wq_b'].reshape(1, -1),
            p['gamma_w1'].T, p['gamma_b1'].reshape(1, -1),
            p['pa_w1'].T, p['pa_b1'].reshape(1, -1)]

    def spec(a):
        if a.ndim == 3:
            return pl.BlockSpec((1,) + a.shape[1:], lambda b, t: (b, 0, 0))
        return pl.BlockSpec(a.shape, lambda b, t: (0,) * a.ndim)

    return pl.pallas_call(
        _stats2_body,
        grid=(B, M // QT),
        in_specs=[
            pl.BlockSpec((1, QTK, 16), lambda b, t: (b, t, 0)),
            pl.BlockSpec((1, QT, 3), lambda b, t: (b, t, 0)),
            pl.BlockSpec((1, QT, DIM), lambda b, t: (b, t, 0)),
            pl.BlockSpec((1, QTK, HIDDEN), lambda b, t: (b, t, 0)),
        ] + [spec(a) for a in args[4:]],
        out_specs=[pl.BlockSpec((1, 2, HIDDEN), lambda b, t: (b, 0, 0))] * 2
        + [
            pl.BlockSpec((1, QTK, HIDDEN), lambda b, t: (b, t, 0)),
            pl.BlockSpec((1, QTK, DIM), lambda b, t: (b, t, 0)),
            pl.BlockSpec((1, QTK, HIDDEN), lambda b, t: (b, t, 0)),
            pl.BlockSpec((1, QTK, HIDDEN), lambda b, t: (b, t, 0)),
        ],
        out_shape=[jax.ShapeDtypeStruct((B, 2, HIDDEN), jnp.float32)] * 2
        + [
            jax.ShapeDtypeStruct((B, MK, HIDDEN), jnp.float32),
            jax.ShapeDtypeStruct((B, MK, DIM), jnp.float32),
            jax.ShapeDtypeStruct((B, MK, HIDDEN), jnp.float32),
            jax.ShapeDtypeStruct((B, MK, HIDDEN), jnp.float32),
        ],
    )(*args)


def _softmax_k(a):
    a3 = a.reshape(QT, K, a.shape[1])
    amax = jnp.max(a3, axis=1, keepdims=True)
    ae = jnp.exp(a3 - amax)
    return ae / jnp.sum(ae, axis=1, keepdims=True)


def _final_body(a1_ref, pa1_ref, pos_ref, e2_ref, valg_ref, qf_ref,
                msk_ref, embd_ref,
                gsc_ref, gsh_ref, gw2_ref, gb2_ref, postw_ref, postb_ref,
                asc_ref, ash_ref, aw2_ref, ab2_ref, emb_ref):
    msk = msk_ref[0] > 0
    pos = pos_ref[0]
    a = jax.nn.relu(a1_ref[0] * gsc_ref[0] + gsh_ref[0])
    a = jnp.dot(a, gw2_ref[...],
                preferred_element_type=jnp.float32) + gb2_ref[...]
    a = a * RSQRTH
    a = jnp.where(msk, a, -FMAX)
    aw = _softmax_k(a)
    vpe = (valg_ref[0] + pos).reshape(QT, K, HIDDEN)
    res = jnp.sum(aw * vpe, axis=1)
    emb_f = jnp.dot(res, postw_ref[...],
                    preferred_element_type=jnp.float32) + postb_ref[...]
    emb_f = emb_f + qf_ref[0]
    pa = jax.nn.relu(pa1_ref[0] * asc_ref[0] + ash_ref[0])
    pa2 = jnp.dot(pa, aw2_ref[...],
                  preferred_element_type=jnp.float32) + ab2_ref[...]
    pa2 = jnp.where(msk, pa2, -FMAX)
    pw = _softmax_k(pa2)
    e3 = e2_ref[0].reshape(QT, K, DIM)
    emb_p = jnp.sum(e3 * pw, axis=1)
    emb_ref[0] = emb_f + emb_p + embd_ref[0]


def _final_pallas(a1, pa1, pos, e2, valg, qf, maskrow, embd, gcoef, acoef, p):
    args = [a1, pa1, pos, e2, valg, qf, maskrow, embd,
            gcoef[0], gcoef[1], p['gamma_w2'].T, p['gamma_b2'].reshape(1, -1),
            p['post_w'].T, p['post_b'].reshape(1, -1),
            acoef[0], acoef[1], p['pa_w2'].T, p['pa_b2'].reshape(1, -1)]

    def spec(a):
        if a.ndim == 3:
            return pl.BlockSpec((1,) + a.shape[1:], lambda b, t: (b, 0, 0))
        return pl.BlockSpec(a.shape, lambda b, t: (0,) * a.ndim)

    return pl.pallas_call(
        _final_body,
        grid=(B, M // QT),
        in_specs=[
            pl.BlockSpec((1, QTK, HIDDEN), lambda b, t: (b, t, 0)),
            pl.BlockSpec((1, QTK, HIDDEN), lambda b, t: (b, t, 0)),
            pl.BlockSpec((1, QTK, HIDDEN), lambda b, t: (b, t, 0)),
            pl.BlockSpec((1, QTK, DIM), lambda b, t: (b, t, 0)),
            pl.BlockSpec((1, QTK, HIDDEN), lambda b, t: (b, t, 0)),
            pl.BlockSpec((1, QT, DIM), lambda b, t: (b, t, 0)),
            pl.BlockSpec((1, QTK, 1), lambda b, t: (b, t, 0)),
            pl.BlockSpec((1, QT, DIM), lambda b, t: (b, t, 0)),
        ] + [spec(a) for a in args[8:]],
        out_specs=pl.BlockSpec((1, QT, DIM), lambda b, t: (b, t, 0)),
        out_shape=jax.ShapeDtypeStruct((B, M, DIM), jnp.float32),
    )(*args)


def _den_body(dn_ref, w1_ref, b1_ref, g_ref, gb_ref, w2_ref, b2_ref, out_ref):
    e = jnp.dot(dn_ref[0], w1_ref[...],
                preferred_element_type=jnp.float32) + b1_ref[...]
    lane = jax.lax.broadcasted_iota(jnp.int32, (HIDDEN, NGROUPS), 0)
    grp = jax.lax.broadcasted_iota(jnp.int32, (HIDDEN, NGROUPS), 1)
    G = (lane // (HIDDEN // NGROUPS) == grp).astype(jnp.float32)
    s1 = jnp.sum(e, axis=0, keepdims=True)
    s2 = jnp.sum(e * e, axis=0, keepdims=True)
    cnt = float(M * (HIDDEN // NGROUPS))
    gm = jnp.dot(s1, G, preferred_element_type=jnp.float32) / cnt
    gv = jnp.dot(s2, G, preferred_element_type=jnp.float32) / cnt - gm * gm
    meanl = jax.lax.dot_general(gm, G, (((1,), (1,)), ((), ())),
                                preferred_element_type=jnp.float32)
    rstdl = jax.lax.dot_general(jax.lax.rsqrt(gv + 1e-5), G,
                                (((1,), (1,)), ((), ())),
                                preferred_element_type=jnp.float32)
    xn = (e - meanl) * rstdl * g_ref[...] + gb_ref[...]
    xn = jax.nn.relu(xn)
    out_ref[0] = jnp.dot(xn, w2_ref[...],
                         preferred_element_type=jnp.float32) + b2_ref[...]


def _den_pallas(dn_rows, p):
    args = [dn_rows, p['de_w1'].T, p['de_b1'].reshape(1, -1),
            p['de_g'].reshape(1, -1), p['de_gb'].reshape(1, -1),
            p['de_w2'].T, p['de_b2'].reshape(1, -1)]
    wspec = lambda a: pl.BlockSpec(a.shape, lambda b: (0,) * a.ndim)
    return pl.pallas_call(
        _den_body,
        grid=(B,),
        in_specs=[pl.BlockSpec((1, M, 1), lambda b: (b, 0, 0))]
        + [wspec(a) for a in args[1:]],
        out_specs=pl.BlockSpec((1, M, DIM), lambda b: (b, 0, 0)),
        out_shape=jax.ShapeDtypeStruct((B, M, DIM), jnp.float32),
    )(*args)


def _gn_coeffs(st, gamma, beta, cnt):
    cpg = st.shape[-1] // NGROUPS
    s1 = st[:, 0, :].reshape(B, NGROUPS, cpg).sum(-1)
    s2 = st[:, 1, :].reshape(B, NGROUPS, cpg).sum(-1)
    mean = s1 / cnt
    var = s2 / cnt - mean * mean
    rstd = jax.lax.rsqrt(var + 1e-5)
    meanl = jnp.repeat(mean, cpg, axis=-1)
    rstdl = jnp.repeat(rstd, cpg, axis=-1)
    scale = rstdl * gamma[None, :]
    shift = beta[None, :] - meanl * scale
    return scale[:, None, :], shift[:, None, :]


def _conv1d(x, w, b):
    return jnp.einsum('oc,bcn->bon', w, x) + b[None, :, None]


def _conv2d(x, w, b):
    return jnp.einsum('oc,bcmk->bomk', w, x) + b[None, :, None, None]


def _group_norm(x, g, gamma, beta, eps=1e-5):
    b, c = x.shape[0], x.shape[1]
    rest = x.shape[2:]
    xg = x.reshape((b, g, c // g) + rest)
    axes = tuple(range(2, xg.ndim))
    mean = xg.mean(axis=axes, keepdims=True)
    var = xg.var(axis=axes, keepdims=True)
    xg = (xg - mean) / jnp.sqrt(var + eps)
    x = xg.reshape((b, c) + rest)
    shp = (1, c) + (1,) * len(rest)
    return x * gamma.reshape(shp) + beta.reshape(shp)


def _index_points(points, idx):
    return jax.vmap(lambda p, i: p[:, i])(points, idx)


def _knn(k, ref_tr, query_tr):
    d = jnp.sum((query_tr[:, :, None, :] - ref_tr[:, None, :, :]) ** 2, axis=-1)
    _, idx = jax.lax.top_k(-d, k)
    return idx


def _nearby(a_xyzs, b_xyzs, k):
    m = a_xyzs.shape[2]
    a_tr = jnp.transpose(a_xyzs, (0, 2, 1))
    b_tr = jnp.transpose(b_xyzs, (0, 2, 1))
    nn_idx = _knn(1, a_tr, b_tr)
    knn_idx = _knn(k, b_tr, a_tr)
    nn_idx_tr = jnp.transpose(nn_idx, (0, 2, 1))
    expect = jnp.arange(m)[None, :, None]
    actual = _index_points(nn_idx_tr, knn_idx)[:, 0]
    mask = expect == actual
    return mask, knn_idx


def _ptl(q_xyzs, k_xyzs, q_feats, k_feats, v_feats, knn_idx, mask, p):
    knn_xyzs = _index_points(k_xyzs, knn_idx)
    identity = q_feats
    query = _conv1d(q_feats, p['wq_w'], p['wq_b'])
    key = _index_points(_conv1d(k_feats, p['wk_w'], p['wk_b']), knn_idx)
    value = _index_points(_conv1d(v_feats, p['wv_w'], p['wv_b']), knn_idx)
    d = q_xyzs[..., None] - knn_xyzs
    pe = _conv2d(d, p['delta_w1'], p['delta_b1'])
    pe = jax.nn.relu(_group_norm(pe, NGROUPS, p['delta_g'], p['delta_gb']))
    pos_enc = _conv2d(pe, p['delta_w2'], p['delta_b2'])
    a = query[..., None] - key + pos_enc
    a = _conv2d(a, p['gamma_w1'], p['gamma_b1'])
    a = jax.nn.relu(_group_norm(a, NGROUPS, p['gamma_g'], p['gamma_gb']))
    a = _conv2d(a, p['gamma_w2'], p['gamma_b2'])
    a = a / np.sqrt(HIDDEN)
    a = jnp.where(mask[:, None], a, -jnp.finfo(a.dtype).max)
    a = jax.nn.softmax(a, axis=-1)
    res = jnp.einsum('bcmk,bcmk->bcm', a, value + pos_enc)
    return _conv1d(res, p['post_w'], p['post_b']) + identity


def _pos_emb(q_xyzs, k_xyzs, knn_idx, mask, p):
    knn_xyzs = _index_points(k_xyzs, knn_idx)
    offset = knn_xyzs - q_xyzs[..., None]
    dist = jnp.sqrt(jnp.sum(offset ** 2, axis=1, keepdims=True) + 1e-12)
    direction = offset / jnp.maximum(dist, 1e-12)
    local = jnp.concatenate([direction, dist], axis=1)
    e = _conv2d(local, p['pe_w1'], p['pe_b1'])
    e = jax.nn.relu(_group_norm(e, NGROUPS, p['pe_g'], p['pe_gb']))
    e = _conv2d(e, p['pe_w2'], p['pe_b2'])
    a = _conv2d(e, p['pa_w1'], p['pa_b1'])
    a = jax.nn.relu(_group_norm(a, NGROUPS, p['pa_g'], p['pa_gb']))
    a = _conv2d(a, p['pa_w2'], p['pa_b2'])
    a = jnp.where(mask[:, None], a, -jnp.finfo(a.dtype).max)
    a = jax.nn.softmax(a, axis=-1)
    return (e * a).sum(-1)


def _den_emb(dn, p):
    e = _conv1d(dn, p['de_w1'], p['de_b1'])
    e = jax.nn.relu(_group_norm(e, NGROUPS, p['de_g'], p['de_gb']))
    return _conv1d(e, p['de_w2'], p['de_b2'])


def kernel(xyzs, feats, params):
    fps_idx, sampled_xyzs = _fps_pallas(xyzs)
    xyzs_t = jnp.transpose(xyzs, (0, 2, 1))
    knn_idx, nn_idx = _knn_pallas(xyzs_t, sampled_xyzs)
    feats_t = jnp.transpose(feats, (0, 2, 1))
    f2t, keyt, valt = _proj_pallas(feats_t, params)

    offs = (jnp.arange(B, dtype=jnp.int32) * N)[:, None]
    kidx = (knn_idx.reshape(B, MK) + offs).reshape(BMK)
    qidx = (fps_idx + offs).reshape(B * M)
    nnf = jax.lax.bitcast_convert_type(nn_idx, jnp.float32)
    comb = jnp.concatenate(
        [keyt, valt, xyzs_t, nnf[:, :, None],
         jnp.zeros((B, N, CW - 2 * HIDDEN - 4), jnp.float32)],
        axis=-1).reshape(BN, CW)
    combg, qf_f = _sc_gather(comb, f2t.reshape(BN, DIM), kidx, qidx)
    keyg = combg[:, :HIDDEN].reshape(B, MK, HIDDEN)
    valg = combg[:, HIDDEN:2 * HIDDEN].reshape(B, MK, HIDDEN)
    kx = combg[:, 2 * HIDDEN:2 * HIDDEN + 16].reshape(B, MK, 16)
    qf = qf_f.reshape(B, M, DIM)
    actual = jax.lax.bitcast_convert_type(
        combg[:, 2 * HIDDEN + 3], jnp.int32).reshape(B, M, K)
    mask = jnp.arange(M)[None, :, None] == actual
    downsample_num = mask.astype(jnp.float32).sum(-1)
    sxt = jnp.transpose(sampled_xyzs, (0, 2, 1))

    cnt = float((HIDDEN // NGROUPS) * MK)
    st_pe, st_e = _stats1_pallas(kx, sxt, params)
    dcoef = _gn_coeffs(st_pe, params['delta_g'], params['delta_gb'], cnt)
    pcoef = _gn_coeffs(st_e, params['pe_g'], params['pe_gb'], cnt)
    st_a, st_pa, pos, e2, a1, pa1 = _stats2_pallas(
        kx, sxt, qf, keyg, dcoef, pcoef, params)
    gcoef = _gn_coeffs(st_a, params['gamma_g'], params['gamma_gb'], cnt)
    acoef = _gn_coeffs(st_pa, params['pa_g'], params['pa_gb'], cnt)

    maskrow = mask.reshape(B, MK, 1).astype(jnp.float32)
    embd = _den_pallas(downsample_num[:, :, None], params)
    emb = _final_pallas(a1, pa1, pos, e2, valg, qf, maskrow, embd,
                        gcoef, acoef, params)
    embedded_points = jnp.transpose(emb, (0, 2, 1))
    return sampled_xyzs, embedded_points, downsample_num


# speedup vs baseline: 1.0733x; 1.0141x over previous
import functools

import jax
import jax.numpy as jnp
import numpy as np
from jax.experimental import pallas as pl
from jax.experimental.pallas import tpu as pltpu

B = 2
N = 8192
DIM = 256
HIDDEN = 128
NGROUPS = 8
K = 16
RATE = 4
M = N // RATE


def _fps_body(x_ref, idx_ref):
    lane = jax.lax.broadcasted_iota(jnp.int32, (1, N), 1)
    x0 = x_ref[0]
    x1 = x_ref[1]
    idx_ref[0, 0, 0] = 0
    idx_ref[1, 0, 0] = 0

    def body(i, carry):
        d0s, d1s, l0, l1 = carry
        oh0 = jnp.where(lane == l0, 1.0, 0.0)
        oh1 = jnp.where(lane == l1, 1.0, 0.0)
        dn = (((1,), (1,)), ((), ()))
        p0 = jax.lax.dot_general(x0, oh0, dn,
                                 preferred_element_type=jnp.float32)
        p1 = jax.lax.dot_general(x1, oh1, dn,
                                 preferred_element_type=jnp.float32)
        d0s = jnp.minimum(d0s, jnp.sum((x0 - p0) ** 2, axis=0, keepdims=True))
        d1s = jnp.minimum(d1s, jnp.sum((x1 - p1) ** 2, axis=0, keepdims=True))
        m0 = jnp.max(d0s)
        m1 = jnp.max(d1s)
        n0 = jnp.min(jnp.where(d0s == m0, lane, N)).astype(jnp.int32)
        n1 = jnp.min(jnp.where(d1s == m1, lane, N)).astype(jnp.int32)
        idx_ref[0, 0, i] = n0
        idx_ref[1, 0, i] = n1
        return d0s, d1s, n0, n1

    dists0 = jnp.full((1, N), jnp.inf, jnp.float32)
    jax.lax.fori_loop(1, M, body,
                      (dists0, dists0, jnp.int32(0), jnp.int32(0)))


def _fps_pallas(xyzs):
    fps_idx = pl.pallas_call(
        _fps_body,
        grid=(1,),
        in_specs=[pl.BlockSpec((B, 3, N), lambda i: (0, 0, 0))],
        out_specs=pl.BlockSpec((B, 1, M), lambda i: (0, 0, 0),
                               memory_space=pltpu.SMEM),
        out_shape=jax.ShapeDtypeStruct((B, 1, M), jnp.int32),
    )(xyzs)
    fps_idx = fps_idx[:, 0, :]
    sampled_xyzs = _index_points(xyzs, fps_idx)
    return fps_idx, sampled_xyzs


TM = 256
NC = 1024
NCHUNKS = N // NC


def _knn_body(xt_ref, s_ref, idx_ref, nnv_ref, nni_ref, d_ref):
    t = pl.program_id(1)
    qq = jax.lax.broadcasted_iota(jnp.int32, (1, TM), 1) + t * TM

    for ci in range(NCHUNKS):
        d = jnp.zeros((NC, TM), jnp.float32)
        for c in range(3):
            xc = xt_ref[0, pl.ds(ci * NC, NC), pl.ds(c, 1)]
            sc = s_ref[0, pl.ds(c, 1), :]
            d = d + (xc - sc) ** 2
        d_ref[pl.ds(ci * NC, NC), :] = d
        rmin = jnp.min(d, axis=1, keepdims=True)
        rarg = jnp.min(jnp.where(d == rmin, qq, M),
                       axis=1, keepdims=True).astype(jnp.int32)

        @pl.when(t == 0)
        def _():
            nnv_ref[0, pl.ds(ci * NC, NC), :] = rmin
            nni_ref[0, pl.ds(ci * NC, NC), :] = rarg

        @pl.when(t != 0)
        def _():
            cur_v = nnv_ref[0, pl.ds(ci * NC, NC), :]
            cur_i = nni_ref[0, pl.ds(ci * NC, NC), :]
            upd = rmin < cur_v
            nnv_ref[0, pl.ds(ci * NC, NC), :] = jnp.where(upd, rmin, cur_v)
            nni_ref[0, pl.ds(ci * NC, NC), :] = jnp.where(upd, rarg, cur_i)

    last_v = jnp.full((1, TM), -jnp.inf, jnp.float32)
    last_i = jnp.full((1, TM), -1, jnp.int32)
    for k in range(K):
        best_v = jnp.full((1, TM), jnp.inf, jnp.float32)
        best_i = jnp.full((1, TM), N, jnp.int32)
        for ci in range(NCHUNKS):
            v = d_ref[pl.ds(ci * NC, NC), :]
            jj = (jax.lax.broadcasted_iota(jnp.int32, (NC, TM), 0)
                  + ci * NC)
            excl = (v < last_v) | ((v == last_v) & (jj <= last_i))
            vv = jnp.where(excl, jnp.inf, v)
            cm = jnp.min(vv, axis=0, keepdims=True)
            ca = jnp.min(jnp.where(vv == cm, jj, N),
                         axis=0, keepdims=True).astype(jnp.int32)
            upd = cm < best_v
            best_i = jnp.where(upd, ca, best_i)
            best_v = jnp.where(upd, cm, best_v)
        idx_ref[0, pl.ds(k, 1), :] = best_i
        last_v, last_i = best_v, best_i


def _knn_pallas(xyzs_t, sampled):
    knn_km, _nnv, nni = pl.pallas_call(
        _knn_body,
        grid=(B, M // TM),
        in_specs=[
            pl.BlockSpec((1, N, 3), lambda b, t: (b, 0, 0)),
            pl.BlockSpec((1, 3, TM), lambda b, t: (b, 0, t)),
        ],
        out_specs=[
            pl.BlockSpec((1, K, TM), lambda b, t: (b, 0, t)),
            pl.BlockSpec((1, N, 1), lambda b, t: (b, 0, 0)),
            pl.BlockSpec((1, N, 1), lambda b, t: (b, 0, 0)),
        ],
        out_shape=[
            jax.ShapeDtypeStruct((B, K, M), jnp.int32),
            jax.ShapeDtypeStruct((B, N, 1), jnp.float32),
            jax.ShapeDtypeStruct((B, N, 1), jnp.int32),
        ],
        scratch_shapes=[pltpu.VMEM((N, TM), jnp.float32)],
    )(xyzs_t, sampled)
    knn_idx = jnp.transpose(knn_km, (0, 2, 1))
    nn_idx = nni[:, :, 0]
    return knn_idx, nn_idx


from jax import lax as _lax
from jax.experimental.pallas import tpu_sc as plsc

MK_ = M * K
BMK = B * MK_
BN = N * B
GC = 128


CW = 384


def _sc_gather(comb, f2t, kidx, qidx):
    info = plsc.get_sparse_core_info()
    nw = info.num_cores * info.num_subcores
    rows_pw = BMK // nw
    nch = rows_pw // GC
    q_pw = (B * M) // nw
    mesh = plsc.VectorSubcoreMesh(core_axis_name="c", subcore_axis_name="s")

    @functools.partial(
        pl.kernel, mesh=mesh,
        out_type=[
            jax.ShapeDtypeStruct((BMK, CW), jnp.float32),
            jax.ShapeDtypeStruct((B * M, DIM), jnp.float32),
        ],
        scratch_types=[
            pltpu.VMEM((GC,), jnp.int32),
            pltpu.VMEM((GC, CW), jnp.float32),
            pltpu.VMEM((q_pw,), jnp.int32),
            pltpu.VMEM((q_pw, DIM), jnp.float32),
            pltpu.SemaphoreType.DMA,
        ],
    )
    def gather_k(comb_h, f2t_h, kidx_h, qidx_h, combg_h, qf_h,
                 idx_v, cbuf, qidx_v, qbuf, sem):
        wid = _lax.axis_index("s") * info.num_cores + _lax.axis_index("c")
        base0 = wid * rows_pw
        for ci in range(nch):
            base = base0 + ci * GC
            pltpu.sync_copy(kidx_h.at[pl.ds(base, GC)], idx_v)
            pltpu.async_copy(comb_h.at[idx_v], cbuf, sem).wait()
            pltpu.sync_copy(cbuf, combg_h.at[pl.ds(base, GC)])
        qbase = wid * q_pw
        pltpu.sync_copy(qidx_h.at[pl.ds(qbase, q_pw)], qidx_v)
        pltpu.async_copy(f2t_h.at[qidx_v], qbuf, sem).wait()
        pltpu.sync_copy(qbuf, qf_h.at[pl.ds(qbase, q_pw)])

    return gather_k(comb, f2t, kidx, qidx)


PT = 512
QT = 256
QTK = QT * K
MK = M * K
FMAX = float(np.finfo(np.float32).max)
RSQRTH = 1.0 / np.sqrt(HIDDEN)


def _proj_body(xt_ref, prew_ref, preb_ref, wk_ref, wkb_ref, wv_ref, wvb_ref,
               f2_ref, key_ref, val_ref):
    x = xt_ref[0]
    f2 = jnp.dot(x, prew_ref[...],
                 preferred_element_type=jnp.float32) + preb_ref[...]
    f2_ref[0] = f2
    key_ref[0] = jnp.dot(f2, wk_ref[...],
                         preferred_element_type=jnp.float32) + wkb_ref[...]
    val_ref[0] = jnp.dot(f2, wv_ref[...],
                         preferred_element_type=jnp.float32) + wvb_ref[...]


def _proj_pallas(feats_t, p):
    w2 = lambda name: p[name].T
    b2 = lambda name: p[name].reshape(1, -1)
    wspec = lambda a: pl.BlockSpec(a.shape, lambda b, t: (0,) * a.ndim)
    args = [feats_t, w2('pre_w'), b2('pre_b'), w2('wk_w'), b2('wk_b'),
            w2('wv_w'), b2('wv_b')]
    return pl.pallas_call(
        _proj_body,
        grid=(B, N // PT),
        in_specs=[pl.BlockSpec((1, PT, DIM), lambda b, t: (b, t, 0))]
        + [wspec(a) for a in args[1:]],
        out_specs=[
            pl.BlockSpec((1, PT, DIM), lambda b, t: (b, t, 0)),
            pl.BlockSpec((1, PT, HIDDEN), lambda b, t: (b, t, 0)),
            pl.BlockSpec((1, PT, HIDDEN), lambda b, t: (b, t, 0)),
        ],
        out_shape=[
            jax.ShapeDtypeStruct((B, N, DIM), jnp.float32),
            jax.ShapeDtypeStruct((B, N, HIDDEN), jnp.float32),
            jax.ShapeDtypeStruct((B, N, HIDDEN), jnp.float32),
        ],
    )(*args)


def _rep_k(x):
    return jnp.reshape(
        jnp.broadcast_to(x[:, None, :], (QT, K, x.shape[1])),
        (QTK, x.shape[1]))


def _geom(kx, sx):
    sxr = _rep_k(sx)
    dg = sxr - kx
    offset = kx - sxr
    dist = jnp.sqrt(jnp.sum(offset ** 2, axis=1, keepdims=True) + 1e-12)
    direction = offset / jnp.maximum(dist, 1e-12)
    local = jnp.concatenate([direction, dist], axis=1)
    return dg, local


def _acc_stats(ref, x, first):
    s = jnp.concatenate([jnp.sum(x, axis=0, keepdims=True),
                         jnp.sum(x * x, axis=0, keepdims=True)], axis=0)

    @pl.when(first)
    def _():
        ref[0] = s

    @pl.when(jnp.logical_not(first))
    def _():
        ref[0] += s


def _stats1_body(kx_ref, sx_ref, dw1_ref, db1_ref, pw1_ref, pb1_ref,
                 stpe_ref, ste_ref):
    first = pl.program_id(1) == 0
    dg, local = _geom(kx_ref[0][:, :3], sx_ref[0])
    pe1 = jnp.dot(dg, dw1_ref[...],
                  preferred_element_type=jnp.float32) + db1_ref[...]
    _acc_stats(stpe_ref, pe1, first)
    e1 = jnp.dot(local, pw1_ref[...],
                 preferred_element_type=jnp.float32) + pb1_ref[...]
    _acc_stats(ste_ref, e1, first)


def _stats1_pallas(kx, sxt, p):
    args = [kx, sxt, p['delta_w1'].T, p['delta_b1'].reshape(1, -1),
            p['pe_w1'].T, p['pe_b1'].reshape(1, -1)]
    wspec = lambda a: pl.BlockSpec(a.shape, lambda b, t: (0,) * a.ndim)
    return pl.pallas_call(
        _stats1_body,
        grid=(B, M // QT),
        in_specs=[
            pl.BlockSpec((1, QTK, 16), lambda b, t: (b, t, 0)),
            pl.BlockSpec((1, QT, 3), lambda b, t: (b, t, 0)),
        ] + [wspec(a) for a in args[2:]],
        out_specs=[pl.BlockSpec((1, 2, HIDDEN), lambda b, t: (b, 0, 0))] * 2,
        out_shape=[jax.ShapeDtypeStruct((B, 2, HIDDEN), jnp.float32)] * 2,
    )(*args)


def _stats2_body(kx_ref, sx_ref, qf_ref, keyg_ref,
                 dw1_ref, db1_ref, dsc_ref, dsh_ref, dw2_ref, db2_ref,
                 pw1_ref, pb1_ref, psc_ref, psh_ref, pw2_ref, pb2_ref,
                 wq_ref, wqb_ref, gw1_ref, gb1_ref, aw1_ref, ab1_ref,
                 sta_ref, stpa_ref, pos_ref, e2_ref, a1_ref, pa1_ref):
    first = pl.program_id(1) == 0
    dg, local = _geom(kx_ref[0][:, :3], sx_ref[0])
    pe1 = jnp.dot(dg, dw1_ref[...],
                  preferred_element_type=jnp.float32) + db1_ref[...]
    pe1 = jax.nn.relu(pe1 * dsc_ref[0] + dsh_ref[0])
    pos = jnp.dot(pe1, dw2_ref[...],
                  preferred_element_type=jnp.float32) + db2_ref[...]
    pos_ref[0] = pos
    e1 = jnp.dot(local, pw1_ref[...],
                 preferred_element_type=jnp.float32) + pb1_ref[...]
    e1 = jax.nn.relu(e1 * psc_ref[0] + psh_ref[0])
    e2 = jnp.dot(e1, pw2_ref[...],
                 preferred_element_type=jnp.float32) + pb2_ref[...]
    e2_ref[0] = e2
    query = jnp.dot(qf_ref[0], wq_ref[...],
                    preferred_element_type=jnp.float32) + wqb_ref[...]
    a0 = _rep_k(query) - keyg_ref[0] + pos
    a1 = jnp.dot(a0, gw1_ref[...],
                 preferred_element_type=jnp.float32) + gb1_ref[...]
    a1_ref[0] = a1
    _acc_stats(sta_ref, a1, first)
    pa1 = jnp.dot(e2, aw1_ref[...],
                  preferred_element_type=jnp.float32) + ab1_ref[...]
    pa1_ref[0] = pa1
    _acc_stats(stpa_ref, pa1, first)


def _stats2_pallas(kx, sxt, qf, keyg, dcoef, pcoef, p):
    args = [kx, sxt, qf, keyg,
            p['delta_w1'].T, p['delta_b1'].reshape(1, -1),
            dcoef[0], dcoef[1], p['delta_w2'].T, p['delta_b2'].reshape(1, -1),
            p['pe_w1'].T, p['pe_b1'].reshape(1, -1),
            pcoef[0], pcoef[1], p['pe_w2'].T, p['pe_b2'].reshape(1, -1),
            p['wq_w'].T, p['wq_b'].reshape(1, -1),
            p['gamma_w1'].T, p['gamma_b1'].reshape(1, -1),
            p['pa_w1'].T, p['pa_b1'].reshape(1, -1)]

    def spec(a):
        if a.ndim == 3:
            return pl.BlockSpec((1,) + a.shape[1:], lambda b, t: (b, 0, 0))
        return pl.BlockSpec(a.shape, lambda b, t: (0,) * a.ndim)

    return pl.pallas_call(
        _stats2_body,
        grid=(B, M // QT),
        in_specs=[
            pl.BlockSpec((1, QTK, 16), lambda b, t: (b, t, 0)),
            pl.BlockSpec((1, QT, 3), lambda b, t: (b, t, 0)),
            pl.BlockSpec((1, QT, DIM), lambda b, t: (b, t, 0)),
            pl.BlockSpec((1, QTK, HIDDEN), lambda b, t: (b, t, 0)),
        ] + [spec(a) for a in args[4:]],
        out_specs=[pl.BlockSpec((1, 2, HIDDEN), lambda b, t: (b, 0, 0))] * 2
        + [
            pl.BlockSpec((1, QTK, HIDDEN), lambda b, t: (b, t, 0)),
            pl.BlockSpec((1, QTK, DIM), lambda b, t: (b, t, 0)),
            pl.BlockSpec((1, QTK, HIDDEN), lambda b, t: (b, t, 0)),
            pl.BlockSpec((1, QTK, HIDDEN), lambda b, t: (b, t, 0)),
        ],
        out_shape=[jax.ShapeDtypeStruct((B, 2, HIDDEN), jnp.float32)] * 2
        + [
            jax.ShapeDtypeStruct((B, MK, HIDDEN), jnp.float32),
            jax.ShapeDtypeStruct((B, MK, DIM), jnp.float32),
            jax.ShapeDtypeStruct((B, MK, HIDDEN), jnp.float32),
            jax.ShapeDtypeStruct((B, MK, HIDDEN), jnp.float32),
        ],
    )(*args)


def _softmax_k(a):
    a3 = a.reshape(QT, K, a.shape[1])
    amax = jnp.max(a3, axis=1, keepdims=True)
    ae = jnp.exp(a3 - amax)
    return ae / jnp.sum(ae, axis=1, keepdims=True)


def _final_body(a1_ref, pa1_ref, pos_ref, e2_ref, valg_ref, qf_ref,
                msk_ref, embd_ref,
                gsc_ref, gsh_ref, gw2_ref, gb2_ref, postw_ref, postb_ref,
                asc_ref, ash_ref, aw2_ref, ab2_ref, emb_ref):
    msk = msk_ref[0] > 0
    pos = pos_ref[0]
    a = jax.nn.relu(a1_ref[0] * gsc_ref[0] + gsh_ref[0])
    a = jnp.dot(a, gw2_ref[...],
                preferred_element_type=jnp.float32) + gb2_ref[...]
    a = a * RSQRTH
    a = jnp.where(msk, a, -FMAX)
    aw = _softmax_k(a)
    vpe = (valg_ref[0] + pos).reshape(QT, K, HIDDEN)
    res = jnp.sum(aw * vpe, axis=1)
    emb_f = jnp.dot(res, postw_ref[...],
                    preferred_element_type=jnp.float32) + postb_ref[...]
    emb_f = emb_f + qf_ref[0]
    pa = jax.nn.relu(pa1_ref[0] * asc_ref[0] + ash_ref[0])
    pa2 = jnp.dot(pa, aw2_ref[...],
                  preferred_element_type=jnp.float32) + ab2_ref[...]
    pa2 = jnp.where(msk, pa2, -FMAX)
    pw = _softmax_k(pa2)
    e3 = e2_ref[0].reshape(QT, K, DIM)
    emb_p = jnp.sum(e3 * pw, axis=1)
    emb_ref[0] = emb_f + emb_p + embd_ref[0]


def _final_pallas(a1, pa1, pos, e2, valg, qf, maskrow, embd, gcoef, acoef, p):
    args = [a1, pa1, pos, e2, valg, qf, maskrow, embd,
            gcoef[0], gcoef[1], p['gamma_w2'].T, p['gamma_b2'].reshape(1, -1),
            p['post_w'].T, p['post_b'].reshape(1, -1),
            acoef[0], acoef[1], p['pa_w2'].T, p['pa_b2'].reshape(1, -1)]

    def spec(a):
        if a.ndim == 3:
            return pl.BlockSpec((1,) + a.shape[1:], lambda b, t: (b, 0, 0))
        return pl.BlockSpec(a.shape, lambda b, t: (0,) * a.ndim)

    return pl.pallas_call(
        _final_body,
        grid=(B, M // QT),
        in_specs=[
            pl.BlockSpec((1, QTK, HIDDEN), lambda b, t: (b, t, 0)),
            pl.BlockSpec((1, QTK, HIDDEN), lambda b, t: (b, t, 0)),
            pl.BlockSpec((1, QTK, HIDDEN), lambda b, t: (b, t, 0)),
            pl.BlockSpec((1, QTK, DIM), lambda b, t: (b, t, 0)),
            pl.BlockSpec((1, QTK, HIDDEN), lambda b, t: (b, t, 0)),
            pl.BlockSpec((1, QT, DIM), lambda b, t: (b, t, 0)),
            pl.BlockSpec((1, QTK, 1), lambda b, t: (b, t, 0)),
            pl.BlockSpec((1, QT, DIM), lambda b, t: (b, t, 0)),
        ] + [spec(a) for a in args[8:]],
        out_specs=pl.BlockSpec((1, QT, DIM), lambda b, t: (b, t, 0)),
        out_shape=jax.ShapeDtypeStruct((B, M, DIM), jnp.float32),
    )(*args)


def _den_body(dn_ref, w1_ref, b1_ref, g_ref, gb_ref, w2_ref, b2_ref, out_ref):
    e = jnp.dot(dn_ref[0], w1_ref[...],
                preferred_element_type=jnp.float32) + b1_ref[...]
    lane = jax.lax.broadcasted_iota(jnp.int32, (HIDDEN, NGROUPS), 0)
    grp = jax.lax.broadcasted_iota(jnp.int32, (HIDDEN, NGROUPS), 1)
    G = (lane // (HIDDEN // NGROUPS) == grp).astype(jnp.float32)
    s1 = jnp.sum(e, axis=0, keepdims=True)
    s2 = jnp.sum(e * e, axis=0, keepdims=True)
    cnt = float(M * (HIDDEN // NGROUPS))
    gm = jnp.dot(s1, G, preferred_element_type=jnp.float32) / cnt
    gv = jnp.dot(s2, G, preferred_element_type=jnp.float32) / cnt - gm * gm
    meanl = jax.lax.dot_general(gm, G, (((1,), (1,)), ((), ())),
                                preferred_element_type=jnp.float32)
    rstdl = jax.lax.dot_general(jax.lax.rsqrt(gv + 1e-5), G,
                                (((1,), (1,)), ((), ())),
                                preferred_element_type=jnp.float32)
    xn = (e - meanl) * rstdl * g_ref[...] + gb_ref[...]
    xn = jax.nn.relu(xn)
    out_ref[0] = jnp.dot(xn, w2_ref[...],
                         preferred_element_type=jnp.float32) + b2_ref[...]


def _den_pallas(dn_rows, p):
    args = [dn_rows, p['de_w1'].T, p['de_b1'].reshape(1, -1),
            p['de_g'].reshape(1, -1), p['de_gb'].reshape(1, -1),
            p['de_w2'].T, p['de_b2'].reshape(1, -1)]
    wspec = lambda a: pl.BlockSpec(a.shape, lambda b: (0,) * a.ndim)
    return pl.pallas_call(
        _den_body,
        grid=(B,),
        in_specs=[pl.BlockSpec((1, M, 1), lambda b: (b, 0, 0))]
        + [wspec(a) for a in args[1:]],
        out_specs=pl.BlockSpec((1, M, DIM), lambda b: (b, 0, 0)),
        out_shape=jax.ShapeDtypeStruct((B, M, DIM), jnp.float32),
    )(*args)


def _gn_coeffs(st, gamma, beta, cnt):
    cpg = st.shape[-1] // NGROUPS
    s1 = st[:, 0, :].reshape(B, NGROUPS, cpg).sum(-1)
    s2 = st[:, 1, :].reshape(B, NGROUPS, cpg).sum(-1)
    mean = s1 / cnt
    var = s2 / cnt - mean * mean
    rstd = jax.lax.rsqrt(var + 1e-5)
    meanl = jnp.repeat(mean, cpg, axis=-1)
    rstdl = jnp.repeat(rstd, cpg, axis=-1)
    scale = rstdl * gamma[None, :]
    shift = beta[None, :] - meanl * scale
    return scale[:, None, :], shift[:, None, :]


def _index_points(points, idx):
    return jax.vmap(lambda p, i: p[:, i])(points, idx)


def kernel(xyzs, feats, params):
    fps_idx, sampled_xyzs = _fps_pallas(xyzs)
    xyzs_t = jnp.transpose(xyzs, (0, 2, 1))
    knn_idx, nn_idx = _knn_pallas(xyzs_t, sampled_xyzs)
    feats_t = jnp.transpose(feats, (0, 2, 1))
    f2t, keyt, valt = _proj_pallas(feats_t, params)

    offs = (jnp.arange(B, dtype=jnp.int32) * N)[:, None]
    kidx = (knn_idx.reshape(B, MK) + offs).reshape(BMK)
    qidx = (fps_idx + offs).reshape(B * M)
    nnf = jax.lax.bitcast_convert_type(nn_idx, jnp.float32)
    comb = jnp.concatenate(
        [keyt, valt, xyzs_t, nnf[:, :, None],
         jnp.zeros((B, N, CW - 2 * HIDDEN - 4), jnp.float32)],
        axis=-1).reshape(BN, CW)
    combg, qf_f = _sc_gather(comb, f2t.reshape(BN, DIM), kidx, qidx)
    keyg = combg[:, :HIDDEN].reshape(B, MK, HIDDEN)
    valg = combg[:, HIDDEN:2 * HIDDEN].reshape(B, MK, HIDDEN)
    kx = combg[:, 2 * HIDDEN:2 * HIDDEN + 16].reshape(B, MK, 16)
    qf = qf_f.reshape(B, M, DIM)
    actual = jax.lax.bitcast_convert_type(
        combg[:, 2 * HIDDEN + 3], jnp.int32).reshape(B, M, K)
    mask = jnp.arange(M)[None, :, None] == actual
    downsample_num = mask.astype(jnp.float32).sum(-1)
    sxt = jnp.transpose(sampled_xyzs, (0, 2, 1))

    cnt = float((HIDDEN // NGROUPS) * MK)
    st_pe, st_e = _stats1_pallas(kx, sxt, params)
    dcoef = _gn_coeffs(st_pe, params['delta_g'], params['delta_gb'], cnt)
    pcoef = _gn_coeffs(st_e, params['pe_g'], params['pe_gb'], cnt)
    st_a, st_pa, pos, e2, a1, pa1 = _stats2_pallas(
        kx, sxt, qf, keyg, dcoef, pcoef, params)
    gcoef = _gn_coeffs(st_a, params['gamma_g'], params['gamma_gb'], cnt)
    acoef = _gn_coeffs(st_pa, params['pa_g'], params['pa_gb'], cnt)

    maskrow = mask.reshape(B, MK, 1).astype(jnp.float32)
    embd = _den_pallas(downsample_num[:, :, None], params)
    emb = _final_pallas(a1, pa1, pos, e2, valg, qf, maskrow, embd,
                        gcoef, acoef, params)
    embedded_points = jnp.transpose(emb, (0, 2, 1))
    return sampled_xyzs, embedded_points, downsample_num
